# speedup vs baseline: 1.7487x; 1.7487x over previous
import functools

import jax
import jax.numpy as jnp
from jax import lax
from jax.experimental import pallas as pl
from jax.experimental.pallas import tpu as pltpu
from jax.experimental.pallas import tpu_sc as plsc

NUM_CORES = 2
NUM_SUBCORES = 16
NUM_WORKERS = NUM_CORES * NUM_SUBCORES
CHUNK = 8


@functools.lru_cache(maxsize=None)
def _build(ntok, vocab, hidden):
    tok_per_w = ntok // NUM_WORKERS
    nchunk = tok_per_w // CHUNK
    assert ntok % NUM_WORKERS == 0 and tok_per_w % CHUNK == 0
    assert nchunk % 2 == 0 and nchunk >= 4

    mesh = plsc.VectorSubcoreMesh(
        core_axis_name="c", subcore_axis_name="s",
        num_cores=NUM_CORES, num_subcores=NUM_SUBCORES)

    @functools.partial(
        pl.kernel,
        out_type=jax.ShapeDtypeStruct((ntok, hidden), jnp.float32),
        mesh=mesh,
        scratch_types=[
            pltpu.VMEM((tok_per_w,), jnp.int32),
            pltpu.VMEM((CHUNK, hidden), jnp.float32),
            pltpu.VMEM((CHUNK, hidden), jnp.float32),
            pltpu.SemaphoreType.DMA,
            pltpu.SemaphoreType.DMA,
        ],
    )
    def emb(ids_hbm, table_hbm, out_hbm, idx_v, buf0, buf1, sem0, sem1):
        wid = lax.axis_index("s") * NUM_CORES + lax.axis_index("c")
        base = wid * tok_per_w

        pltpu.sync_copy(ids_hbm.at[pl.ds(base, tok_per_w)], idx_v)

        def gather(c, buf, sem):
            off = pl.multiple_of(c * CHUNK, CHUNK)
            return pltpu.async_copy(
                table_hbm.at[idx_v.at[pl.ds(off, CHUNK)]], buf, sem)

        def store(c, buf):
            pltpu.sync_copy(buf, out_hbm.at[pl.ds(base + c * CHUNK, CHUNK)])

        def wait(buf, sem):
            pltpu.make_async_copy(
                table_hbm.at[idx_v.at[pl.ds(0, CHUNK)]], buf, sem).wait()

        gather(0, buf0, sem0)
        gather(1, buf1, sem1)

        def body(i2):
            c = i2 * 2
            wait(buf0, sem0)
            store(c, buf0)
            gather(c + 2, buf0, sem0)
            wait(buf1, sem1)
            store(c + 1, buf1)
            gather(c + 3, buf1, sem1)

        pl.loop(0, nchunk // 2 - 1)(body)

        wait(buf0, sem0)
        store(nchunk - 2, buf0)
        wait(buf1, sem1)
        store(nchunk - 1, buf1)

    return emb


def kernel(input_ids, word_embeddings):
    b, s = input_ids.shape
    vocab, hidden = word_embeddings.shape
    ids = input_ids.reshape(-1).astype(jnp.int32)
    out = _build(b * s, vocab, hidden)(ids, word_embeddings)
    return out.reshape(b, s, hidden)


# speedup vs baseline: 1.7510x; 1.0013x over previous
import functools

import jax
import jax.numpy as jnp
from jax import lax
from jax.experimental import pallas as pl
from jax.experimental.pallas import tpu as pltpu
from jax.experimental.pallas import tpu_sc as plsc

NUM_CORES = 2
NUM_SUBCORES = 16
NUM_WORKERS = NUM_CORES * NUM_SUBCORES
CHUNK = 4
NBUF = 4


@functools.lru_cache(maxsize=None)
def _build(ntok, vocab, hidden):
    tok_per_w = ntok // NUM_WORKERS
    nchunk = tok_per_w // CHUNK
    assert ntok % NUM_WORKERS == 0 and tok_per_w % CHUNK == 0
    assert nchunk % NBUF == 0 and nchunk >= 2 * NBUF

    mesh = plsc.VectorSubcoreMesh(
        core_axis_name="c", subcore_axis_name="s",
        num_cores=NUM_CORES, num_subcores=NUM_SUBCORES)

    @functools.partial(
        pl.kernel,
        out_type=jax.ShapeDtypeStruct((ntok, hidden), jnp.float32),
        mesh=mesh,
        scratch_types=[
            pltpu.VMEM((nchunk, CHUNK), jnp.int32),
            [pltpu.VMEM((CHUNK, hidden), jnp.float32) for _ in range(NBUF)],
            [pltpu.SemaphoreType.DMA for _ in range(NBUF)],
            [pltpu.SemaphoreType.DMA for _ in range(NBUF)],
        ],
    )
    def emb(ids_hbm, table_hbm, out_hbm, idx_v, bufs, gsems, ssems):
        wid = lax.axis_index("s") * NUM_CORES + lax.axis_index("c")
        base = wid * tok_per_w

        pltpu.sync_copy(ids_hbm.at[wid], idx_v)

        def gather(c, b):
            pltpu.async_copy(table_hbm.at[idx_v.at[c]], bufs[b], gsems[b])

        def wait_gather(b):
            pltpu.make_async_copy(
                table_hbm.at[idx_v.at[0]], bufs[b], gsems[b]).wait()

        def store(c, b):
            pltpu.async_copy(
                bufs[b], out_hbm.at[pl.ds(base + c * CHUNK, CHUNK)], ssems[b])

        def wait_store(b):
            pltpu.make_async_copy(
                bufs[b], out_hbm.at[pl.ds(base, CHUNK)], ssems[b]).wait()

        gather(0, 0)
        gather(1, 1)
        wait_gather(0)
        store(0, 0)
        gather(2, 2)
        wait_gather(1)
        store(1, 1)
        gather(3, 3)

        def body(j):
            c0 = j * NBUF + 2
            for u in range(NBUF):
                b = (2 + u) % NBUF
                wait_gather(b)
                store(c0 + u, b)
                bg = (2 + u + 2) % NBUF
                wait_store(bg)
                gather(c0 + u + 2, bg)

        pl.loop(0, (nchunk - NBUF) // NBUF)(body)

        b = (nchunk - 2) % NBUF
        wait_gather(b)
        store(nchunk - 2, b)
        b = (nchunk - 1) % NBUF
        wait_gather(b)
        store(nchunk - 1, b)
        for b in range(NBUF):
            wait_store(b)

    return emb


def kernel(input_ids, word_embeddings):
    b, s = input_ids.shape
    vocab, hidden = word_embeddings.shape
    ntok = b * s
    tok_per_w = ntok // NUM_WORKERS
    ids = input_ids.reshape(
        NUM_WORKERS, tok_per_w // CHUNK, CHUNK).astype(jnp.int32)
    out = _build(ntok, vocab, hidden)(ids, word_embeddings)
    return out.reshape(b, s, hidden)
